# baseline (speedup 1.0000x reference)
import jax
import jax.numpy as jnp
from jax.experimental import pallas as pl
from jax.experimental.pallas import tpu as pltpu

N, C = 4096, 256
H = N // 2
BM = 128
NB = 16
SM = 1024
SPB = SM // (N // NB)
EPS = 1e-5


def _body(x_ref, la_ref, lb_ref, w_ref, b_ref, g_ref, be_ref, out_ref,
          l_bf, xh, tx1, tx2, acc, obuf):
    s = pl.program_id(0)
    i = pl.program_id(1)

    @pl.when((s == 0) & (i == 0))
    def _bn():
        xv = x_ref[...]
        mean = jnp.mean(xv, axis=0, keepdims=True)
        var = jnp.mean((xv - mean) ** 2, axis=0, keepdims=True)
        xhv = (xv - mean) / jnp.sqrt(var + EPS) * g_ref[...] + be_ref[...]
        xh[...] = xhv.astype(jnp.bfloat16)

    @pl.when(s == 0)
    def _s0():
        for half, l_in in ((0, la_ref), (1, lb_ref)):
            rows = pl.ds(half * H + i * BM, BM)
            lb = l_in[...].astype(jnp.bfloat16)
            l_bf[rows, :] = lb
            t1 = jnp.dot(lb, xh[...], preferred_element_type=jnp.float32)
            t1_bf = t1.astype(jnp.bfloat16)
            tx1[rows, :] = t1_bf
            acc[rows, :] = (
                jnp.dot(xh[rows, :], w_ref[0],
                        preferred_element_type=jnp.float32)
                + jnp.dot(t1_bf, w_ref[1],
                          preferred_element_type=jnp.float32)
            ).astype(jnp.bfloat16)

    crows = pl.ds((i // SPB) * SM, SM)

    @pl.when((s == 1) & (i % SPB == 0))
    def _s1():
        t2 = (2.0 * jnp.dot(l_bf[crows, :], tx1[...],
                            preferred_element_type=jnp.float32)
              - xh[crows, :].astype(jnp.float32))
        t2_bf = t2.astype(jnp.bfloat16)
        tx2[crows, :] = t2_bf
        acc[crows, :] = (acc[crows, :].astype(jnp.float32) + jnp.dot(
            t2_bf, w_ref[2], preferred_element_type=jnp.float32)
        ).astype(jnp.bfloat16)

    @pl.when(s == 2)
    def _s2():
        @pl.when(i % SPB == 0)
        def _big():
            t3 = (2.0 * jnp.dot(l_bf[crows, :], tx2[...],
                                preferred_element_type=jnp.float32)
                  - tx1[crows, :].astype(jnp.float32))
            o = acc[crows, :].astype(jnp.float32) + jnp.dot(
                t3.astype(jnp.bfloat16), w_ref[3],
                preferred_element_type=jnp.float32) + b_ref[...]
            obuf[...] = jnp.maximum(o, 0.0)

        ob = N // NB
        out_ref[...] = obuf[pl.ds((i % SPB) * ob, ob), :]


def kernel(x, laplacian, W, bias, gamma, beta):
    b2 = bias.reshape(1, C)
    g2 = gamma.reshape(1, C)
    be2 = beta.reshape(1, C)
    w_bf = W.astype(jnp.bfloat16)
    ob = N // NB
    return pl.pallas_call(
        _body,
        grid=(3, NB),
        in_specs=[
            pl.BlockSpec((N, C), lambda s, i: (0, 0)),
            pl.BlockSpec((BM, N),
                         lambda s, i: (jnp.where(s == 0, i, 0), 0)),
            pl.BlockSpec((BM, N),
                         lambda s, i: (jnp.where(s == 0, NB + i, NB), 0)),
            pl.BlockSpec((4, C, C), lambda s, i: (0, 0, 0)),
            pl.BlockSpec((1, C), lambda s, i: (0, 0)),
            pl.BlockSpec((1, C), lambda s, i: (0, 0)),
            pl.BlockSpec((1, C), lambda s, i: (0, 0)),
        ],
        out_specs=pl.BlockSpec((ob, C), lambda s, i: (i, 0)),
        out_shape=jax.ShapeDtypeStruct((N, C), jnp.float32),
        scratch_shapes=[
            pltpu.VMEM((N, N), jnp.bfloat16),
            pltpu.VMEM((N, C), jnp.bfloat16),
            pltpu.VMEM((N, C), jnp.bfloat16),
            pltpu.VMEM((N, C), jnp.bfloat16),
            pltpu.VMEM((N, C), jnp.bfloat16),
            pltpu.VMEM((SM, C), jnp.float32),
        ],
    )(x, laplacian, laplacian, w_bf, b2, g2, be2)
